# baseline (speedup 1.0000x reference)
import functools

import jax
import jax.numpy as jnp
from jax import lax
from jax.experimental import pallas as pl
from jax.experimental.pallas import tpu as pltpu
from jax.experimental.pallas import tpu_sc as plsc

N = 10000
E = 320000
H = 128

NC = 2
NS = 16
NW = NC * NS
CHUNK = 128
CHUNKS_PER_W = 80
N_CHUNKS = CHUNKS_PER_W * NW
E_PAD = N_CHUNKS * CHUNK
NBUF = 2
PHASES = 2
CPP = CHUNKS_PER_W // PHASES
NGROUPS = CPP // NBUF
ROWS_PER_TILE = 624
TAIL0 = ROWS_PER_TILE * NS
TAIL = N - TAIL0
N_ACC = N + 8


def _encode_body(x_ref, w1_ref, b1_ref, w2_ref, b2_ref, ns_ref, s_ref):
    h = jnp.dot(x_ref[...], w1_ref[...], preferred_element_type=jnp.float32)
    h = jnp.maximum(h + b1_ref[...], 0.0)
    ns = jnp.dot(h, w2_ref[...], preferred_element_type=jnp.float32) + b2_ref[...]
    ns_ref[...] = ns
    s_ref[...] = jnp.sum(ns).reshape(1, 1)


def _encode(x, w1, b1, w2, b2):
    return pl.pallas_call(
        _encode_body,
        out_shape=(
            jax.ShapeDtypeStruct((N, H), jnp.float32),
            jax.ShapeDtypeStruct((1, 1), jnp.float32),
        ),
    )(x, w1, b1, w2, b2)


def _edge_agg_body(ns_hbm, src_hbm, dst_hbm, z2_hbm, z1_hbm,
                   agg_out, deg_out,
                   src_v, dst_v, rows_v, ones_v, agg_sh, deg_sh,
                   gsem, ssem, dsem):
    cid = lax.axis_index("c")
    sid = lax.axis_index("s")
    wid = sid * NC + cid
    row0 = wid * CHUNKS_PER_W

    r0 = sid * ROWS_PER_TILE
    pltpu.sync_copy(z2_hbm.at[pl.ds(r0, ROWS_PER_TILE)],
                    agg_sh.at[pl.ds(r0, ROWS_PER_TILE)])

    @pl.when(sid == NS - 1)
    def _():
        pltpu.sync_copy(z2_hbm.at[pl.ds(TAIL0, TAIL)],
                        agg_sh.at[pl.ds(TAIL0, TAIL)])

    @pl.when(sid == 0)
    def _():
        pltpu.sync_copy(z1_hbm, deg_sh)

    for j in range(CHUNK // 16):
        ones_v[pl.ds(16 * j, 16)] = jnp.ones((16,), jnp.float32)

    plsc.subcore_barrier()

    def _gather(k, b):
        return pltpu.make_async_copy(ns_hbm.at[src_v.at[k]], rows_v[b], gsem[b])

    def _scat_rows(k, b):
        return pltpu.make_async_copy(rows_v[b], agg_sh.at[dst_v.at[k]], ssem[b])

    def _scat_deg(k, b):
        return pltpu.make_async_copy(ones_v, deg_sh.at[dst_v.at[k]], dsem[b])

    for phase in range(PHASES):
        rp = row0 + phase * CPP
        pltpu.sync_copy(src_hbm.at[pl.ds(rp, CPP)], src_v)
        pltpu.sync_copy(dst_hbm.at[pl.ds(rp, CPP)], dst_v)

        for b in range(NBUF):
            _gather(b, b).start()

        def body(g, _):
            for b in range(NBUF):
                k = g * NBUF + b
                _gather(k, b).wait()
                _scat_rows(k, b).start(add=True)
                _scat_deg(k, b).start(add=True)

                @pl.when(g < NGROUPS - 1)
                def _():
                    _scat_rows(k, b).wait()
                    _scat_deg(k, b).wait()
                    _gather(k + NBUF, b).start()

            return ()

        lax.fori_loop(0, NGROUPS, body, ())

        last = (NGROUPS - 1) * NBUF
        for b in range(NBUF):
            _scat_rows(last + b, b).wait()
            _scat_deg(last + b, b).wait()

    plsc.subcore_barrier()

    pltpu.sync_copy(agg_sh.at[pl.ds(r0, ROWS_PER_TILE)],
                    agg_out.at[cid, pl.ds(r0, ROWS_PER_TILE)])

    @pl.when(sid == NS - 1)
    def _():
        pltpu.sync_copy(agg_sh.at[pl.ds(TAIL0, TAIL)],
                        agg_out.at[cid, pl.ds(TAIL0, TAIL)])

    @pl.when(sid == 0)
    def _():
        pltpu.sync_copy(deg_sh, deg_out.at[cid])


_edge_agg = functools.partial(
    pl.kernel,
    out_type=(
        jax.ShapeDtypeStruct((NC, N, H), jnp.float32),
        jax.ShapeDtypeStruct((NC, N_ACC), jnp.float32),
    ),
    mesh=plsc.VectorSubcoreMesh(core_axis_name="c", subcore_axis_name="s",
                                num_cores=NC, num_subcores=NS),
    scratch_types=[
        pltpu.VMEM((CPP, CHUNK), jnp.int32),
        pltpu.VMEM((CPP, CHUNK), jnp.int32),
        [pltpu.VMEM((CHUNK, H), jnp.float32)] * NBUF,
        pltpu.VMEM((CHUNK,), jnp.float32),
        pltpu.VMEM_SHARED((N_ACC, H), jnp.float32),
        pltpu.VMEM_SHARED((N_ACC,), jnp.float32),
        [pltpu.SemaphoreType.DMA] * NBUF,
        [pltpu.SemaphoreType.DMA] * NBUF,
        [pltpu.SemaphoreType.DMA] * NBUF,
    ],
)(_edge_agg_body)


def _finish_body(agg_ref, deg_ref, s_ref, wg_ref, bg_ref, wphi_ref, bphi_ref,
                 we1_ref, be1_ref, we2_ref, be2_ref,
                 phi_ref, ent_ref, zk_ref, gs_ref):
    agg = agg_ref[0] + agg_ref[1]
    deg = (deg_ref[0] + deg_ref[1])[:N]
    agg = agg / jnp.clip(deg, 1.0)[:, None]
    g = jnp.dot(agg, wg_ref[...], preferred_element_type=jnp.float32)
    g = jnp.maximum(g + bg_ref[...], 0.0)
    gs = (jnp.sum(g, axis=0, keepdims=True) / N)
    gs_ref[...] = gs
    phi_ref[...] = jnp.dot(gs, wphi_ref[...],
                           preferred_element_type=jnp.float32) + bphi_ref[...]
    e = jnp.maximum(jnp.dot(gs, we1_ref[...],
                            preferred_element_type=jnp.float32) + be1_ref[...], 0.0)
    logit = jnp.dot(e, we2_ref[...],
                    preferred_element_type=jnp.float32) + be2_ref[...]
    ent_ref[...] = 1.0 / (1.0 + jnp.exp(-logit))
    zk_ref[...] = jnp.tanh(s_ref[...] / N)


def _finish(agg2, deg2, s, wg, bg, wphi, bphi, we1, be1, we2, be2):
    return pl.pallas_call(
        _finish_body,
        out_shape=(
            jax.ShapeDtypeStruct((1, 1), jnp.float32),
            jax.ShapeDtypeStruct((1, 1), jnp.float32),
            jax.ShapeDtypeStruct((1, 1), jnp.float32),
            jax.ShapeDtypeStruct((1, H), jnp.float32),
        ),
    )(agg2, deg2, s, wg, bg, wphi, bphi, we1, be1, we2, be2)


def kernel(neural_data, edge_index, W1, b1, W2, b2, Wg, bg, Wphi, bphi,
           We1, be1, We2, be2):
    ns, s = _encode(neural_data, W1, b1.reshape(1, H), W2, b2.reshape(1, H))
    npad = E_PAD - E
    src = jnp.concatenate([edge_index[0], jnp.zeros((npad,), jnp.int32)])
    dst = jnp.concatenate([edge_index[1], jnp.full((npad,), N, jnp.int32)])
    src2 = src.reshape(N_CHUNKS, CHUNK)
    dst2 = dst.reshape(N_CHUNKS, CHUNK)
    z2 = jnp.zeros((N, H), jnp.float32)
    z1 = jnp.zeros((N_ACC,), jnp.float32)
    agg2, deg2 = _edge_agg(ns, src2, dst2, z2, z1)
    phi, ent, zk, gs = _finish(agg2, deg2, s, Wg, bg.reshape(1, H),
                               Wphi, bphi.reshape(1, 1),
                               We1, be1.reshape(1, 64), We2, be2.reshape(1, 1))
    return (phi, ent, zk[0, 0], gs)


# speedup vs baseline: 3.0976x; 3.0976x over previous
import functools

import jax
import jax.numpy as jnp
from jax import lax
from jax.experimental import pallas as pl
from jax.experimental.pallas import tpu as pltpu
from jax.experimental.pallas import tpu_sc as plsc

N = 10000
E = 320000
H = 128

NC = 2
NS = 16
NW = NC * NS
CHUNK = 128
CHUNKS_PER_W = 80
N_CHUNKS = CHUNKS_PER_W * NW
N_REAL = E // CHUNK
E_PAD = N_CHUNKS * CHUNK
NBUF = 2
PHASES = 2
CPP = CHUNKS_PER_W // PHASES
NGROUPS = CPP // NBUF
ROWS_PER_TILE = 624
TAIL0 = ROWS_PER_TILE * NS
TAIL = N - TAIL0
N_ACC = N + 8


def _encode_body(x_ref, w1_ref, b1_ref, w2_ref, b2_ref, ns_ref, s_ref):
    h = jnp.dot(x_ref[...], w1_ref[...], preferred_element_type=jnp.float32)
    h = jnp.maximum(h + b1_ref[...], 0.0)
    ns = jnp.dot(h, w2_ref[...], preferred_element_type=jnp.float32) + b2_ref[...]
    ns_ref[...] = ns
    s_ref[...] = jnp.sum(ns).reshape(1, 1)


def _encode(x, w1, b1, w2, b2):
    return pl.pallas_call(
        _encode_body,
        out_shape=(
            jax.ShapeDtypeStruct((N, H), jnp.float32),
            jax.ShapeDtypeStruct((1, 1), jnp.float32),
        ),
    )(x, w1, b1, w2, b2)


def _edge_agg_body(ns_hbm, src_hbm, dst_hbm, z2_hbm, z1_hbm,
                   agg_out, deg_out,
                   src_v, dst_v, rows_v, ones_v, agg_sh, deg_sh,
                   gsem, ssem, dsem):
    cid = lax.axis_index("c")
    sid = lax.axis_index("s")
    wid = sid * NC + cid
    row0 = wid * CHUNKS_PER_W

    r0 = sid * ROWS_PER_TILE
    pltpu.sync_copy(z2_hbm.at[pl.ds(r0, ROWS_PER_TILE)],
                    agg_sh.at[pl.ds(r0, ROWS_PER_TILE)])

    @pl.when(sid == NS - 1)
    def _():
        pltpu.sync_copy(z2_hbm.at[pl.ds(TAIL0, TAIL)],
                        agg_sh.at[pl.ds(TAIL0, TAIL)])

    @pl.when(sid == 0)
    def _():
        pltpu.sync_copy(z1_hbm, deg_sh)

    for j in range(CHUNK // 16):
        ones_v[pl.ds(16 * j, 16)] = jnp.ones((16,), jnp.float32)

    plsc.subcore_barrier()

    def _gather(k, b):
        return pltpu.make_async_copy(ns_hbm.at[src_v.at[k]], rows_v[b], gsem[b])

    def _scat_rows(k, b):
        return pltpu.make_async_copy(rows_v[b], agg_sh.at[dst_v.at[k]], ssem[b])

    def _scat_deg(k, b):
        return pltpu.make_async_copy(ones_v, deg_sh.at[dst_v.at[k]], dsem[b])

    for phase in range(PHASES):
        rp = row0 + phase * CPP
        pltpu.sync_copy(src_hbm.at[pl.ds(rp, CPP)], src_v)
        pltpu.sync_copy(dst_hbm.at[pl.ds(rp, CPP)], dst_v)

        def real(k):
            return rp + k < N_REAL

        for b in range(NBUF):
            @pl.when(real(b))
            def _():
                _gather(b, b).start()

        def body(g, _):
            for b in range(NBUF):
                k = g * NBUF + b

                @pl.when(real(k))
                def _():
                    _gather(k, b).wait()
                    _scat_rows(k, b).start(add=True)
                    _scat_deg(k, b).start(add=True)

                @pl.when((g < NGROUPS - 1) & real(k + NBUF))
                def _():
                    _scat_rows(k, b).wait()
                    _scat_deg(k, b).wait()
                    _gather(k + NBUF, b).start()

            return ()

        lax.fori_loop(0, NGROUPS, body, ())

        last = (NGROUPS - 1) * NBUF
        for b in range(NBUF):
            @pl.when(real(last + b))
            def _():
                _scat_rows(last + b, b).wait()
                _scat_deg(last + b, b).wait()

    plsc.subcore_barrier()

    pltpu.sync_copy(agg_sh.at[pl.ds(r0, ROWS_PER_TILE)],
                    agg_out.at[cid, pl.ds(r0, ROWS_PER_TILE)])

    @pl.when(sid == NS - 1)
    def _():
        pltpu.sync_copy(agg_sh.at[pl.ds(TAIL0, TAIL)],
                        agg_out.at[cid, pl.ds(TAIL0, TAIL)])

    @pl.when(sid == 0)
    def _():
        pltpu.sync_copy(deg_sh, deg_out.at[cid])


_edge_agg = functools.partial(
    pl.kernel,
    out_type=(
        jax.ShapeDtypeStruct((NC, N, H), jnp.float32),
        jax.ShapeDtypeStruct((NC, N_ACC), jnp.float32),
    ),
    mesh=plsc.VectorSubcoreMesh(core_axis_name="c", subcore_axis_name="s",
                                num_cores=NC, num_subcores=NS),
    scratch_types=[
        pltpu.VMEM((CPP, CHUNK), jnp.int32),
        pltpu.VMEM((CPP, CHUNK), jnp.int32),
        [pltpu.VMEM((CHUNK, H), jnp.float32)] * NBUF,
        pltpu.VMEM((CHUNK,), jnp.float32),
        pltpu.VMEM_SHARED((N_ACC, H), jnp.float32),
        pltpu.VMEM_SHARED((N_ACC,), jnp.float32),
        [pltpu.SemaphoreType.DMA] * NBUF,
        [pltpu.SemaphoreType.DMA] * NBUF,
        [pltpu.SemaphoreType.DMA] * NBUF,
    ],
)(_edge_agg_body)


def _finish_body(agg_ref, deg_ref, s_ref, wg_ref, bg_ref, wphi_ref, bphi_ref,
                 we1_ref, be1_ref, we2_ref, be2_ref,
                 phi_ref, ent_ref, zk_ref, gs_ref):
    agg = agg_ref[0] + agg_ref[1]
    deg = (deg_ref[0] + deg_ref[1])[:N]
    agg = agg / jnp.clip(deg, 1.0)[:, None]
    g = jnp.dot(agg, wg_ref[...], preferred_element_type=jnp.float32)
    g = jnp.maximum(g + bg_ref[...], 0.0)
    gs = (jnp.sum(g, axis=0, keepdims=True) / N)
    gs_ref[...] = gs
    phi_ref[...] = jnp.dot(gs, wphi_ref[...],
                           preferred_element_type=jnp.float32) + bphi_ref[...]
    e = jnp.maximum(jnp.dot(gs, we1_ref[...],
                            preferred_element_type=jnp.float32) + be1_ref[...], 0.0)
    logit = jnp.dot(e, we2_ref[...],
                    preferred_element_type=jnp.float32) + be2_ref[...]
    ent_ref[...] = 1.0 / (1.0 + jnp.exp(-logit))
    zk_ref[...] = jnp.tanh(s_ref[...] / N)


def _finish(agg2, deg2, s, wg, bg, wphi, bphi, we1, be1, we2, be2):
    return pl.pallas_call(
        _finish_body,
        out_shape=(
            jax.ShapeDtypeStruct((1, 1), jnp.float32),
            jax.ShapeDtypeStruct((1, 1), jnp.float32),
            jax.ShapeDtypeStruct((1, 1), jnp.float32),
            jax.ShapeDtypeStruct((1, H), jnp.float32),
        ),
    )(agg2, deg2, s, wg, bg, wphi, bphi, we1, be1, we2, be2)


def kernel(neural_data, edge_index, W1, b1, W2, b2, Wg, bg, Wphi, bphi,
           We1, be1, We2, be2):
    ns, s = _encode(neural_data, W1, b1.reshape(1, H), W2, b2.reshape(1, H))
    npad = E_PAD - E
    src = jnp.concatenate([edge_index[0], jnp.zeros((npad,), jnp.int32)])
    dst = jnp.concatenate([edge_index[1], jnp.full((npad,), N, jnp.int32)])
    src2 = src.reshape(N_CHUNKS, CHUNK)
    dst2 = dst.reshape(N_CHUNKS, CHUNK)
    z2 = jnp.zeros((N, H), jnp.float32)
    z1 = jnp.zeros((N_ACC,), jnp.float32)
    agg2, deg2 = _edge_agg(ns, src2, dst2, z2, z1)
    phi, ent, zk, gs = _finish(agg2, deg2, s, Wg, bg.reshape(1, H),
                               Wphi, bphi.reshape(1, 1),
                               We1, be1.reshape(1, 64), We2, be2.reshape(1, 1))
    return (phi, ent, zk[0, 0], gs)
